# baseline (speedup 1.0000x reference)
import jax
import jax.numpy as jnp
from jax import lax
from jax.experimental import pallas as pl
from jax.experimental.pallas import tpu as pltpu
from jax.experimental.pallas import tpu_sc as plsc

DIM = 64
SCALE = 8.0
LANES = 16

_info = plsc.get_sparse_core_info()
NC, NS = _info.num_cores, _info.num_subcores
NW = NC * NS

IDX_ROW = 128
K = 5
CHUNK = K * IDX_ROW


def _emb_body(table_hbm, idx_hbm, out_hbm, idx_v, rows_v, sem):
    n_rows_total = idx_hbm.shape[0]
    rows_per_w = n_rows_total // NW
    n_chunks = rows_per_w // K
    wid = lax.axis_index("s") * NC + lax.axis_index("c")
    row_base = wid * rows_per_w

    pltpu.sync_copy(idx_hbm.at[pl.ds(row_base, rows_per_w)], idx_v)

    def chunk_body(ci, carry):
        row_off = ci * K
        copies = [
            pltpu.async_copy(
                table_hbm.at[idx_v.at[row_off + j]],
                rows_v.at[pl.ds(j * IDX_ROW, IDX_ROW)],
                sem,
            )
            for j in range(K)
        ]
        for c in copies:
            c.wait()

        def row_body(r, acc):
            for c0 in range(0, DIM, LANES):
                rows_v[r, pl.ds(c0, LANES)] = rows_v[r, pl.ds(c0, LANES)] * SCALE
            return acc

        lax.fori_loop(0, CHUNK, row_body, 0)
        pltpu.sync_copy(
            rows_v, out_hbm.at[pl.ds((row_base + row_off) * IDX_ROW, CHUNK)]
        )
        return carry

    lax.fori_loop(0, n_chunks, chunk_body, 0)


def kernel(x, table):
    B, L = x.shape
    n = B * L
    idx = x.reshape(n // IDX_ROW, IDX_ROW).astype(jnp.int32)
    mesh = plsc.VectorSubcoreMesh(core_axis_name="c", subcore_axis_name="s")
    rows_per_w = (n // IDX_ROW) // NW
    run = pl.kernel(
        _emb_body,
        mesh=mesh,
        compiler_params=pltpu.CompilerParams(use_tc_tiling_on_sc=False),
        out_type=jax.ShapeDtypeStruct((n, DIM), jnp.float32),
        scratch_types=[
            pltpu.VMEM((rows_per_w, IDX_ROW), jnp.int32),
            pltpu.VMEM((CHUNK, DIM), jnp.float32),
            pltpu.SemaphoreType.DMA,
        ],
    )
    out = run(table, idx)
    return out.reshape(B, L, DIM)


# speedup vs baseline: 1.0021x; 1.0021x over previous
import jax
import jax.numpy as jnp
from jax import lax
from jax.experimental import pallas as pl
from jax.experimental.pallas import tpu as pltpu
from jax.experimental.pallas import tpu_sc as plsc

DIM = 64
SCALE = 8.0
LANES = 16

_info = plsc.get_sparse_core_info()
NC, NS = _info.num_cores, _info.num_subcores
NW = NC * NS

IDX_ROW = 128
K = 5
CHUNK = K * IDX_ROW


def _emb_body(table_hbm, idx_hbm, out_hbm, idx_v, rows_v, sem):
    n_rows_total = idx_hbm.shape[0]
    rows_per_w = n_rows_total // NW
    n_chunks = rows_per_w // K
    wid = lax.axis_index("s") * NC + lax.axis_index("c")
    row_base = wid * rows_per_w

    pltpu.sync_copy(idx_hbm.at[pl.ds(row_base, rows_per_w)], idx_v)

    def chunk_body(ci, carry):
        row_off = ci * K
        copies = [
            pltpu.async_copy(
                table_hbm.at[idx_v.at[row_off + j]],
                rows_v.at[pl.ds(j * IDX_ROW, IDX_ROW)],
                sem,
            )
            for j in range(K)
        ]
        for c in copies:
            c.wait()

        def row_body(r, acc):
            for c0 in range(0, DIM, LANES):
                rows_v[r, pl.ds(c0, LANES)] = rows_v[r, pl.ds(c0, LANES)] * SCALE
            return acc

        lax.fori_loop(0, CHUNK, row_body, 0)
        pltpu.sync_copy(
            rows_v, out_hbm.at[pl.ds((row_base + row_off) * IDX_ROW, CHUNK)]
        )
        return carry

    lax.fori_loop(0, n_chunks, chunk_body, 0)


def kernel(x, table):
    B, L = x.shape
    n = B * L
    idx = x.reshape(n // IDX_ROW, IDX_ROW).astype(jnp.int32)
    mesh = plsc.VectorSubcoreMesh(core_axis_name="c", subcore_axis_name="s")
    rows_per_w = (n // IDX_ROW) // NW
    run = pl.kernel(
        _emb_body,
        mesh=mesh,
        compiler_params=pltpu.CompilerParams(
            use_tc_tiling_on_sc=False,
            skip_device_barrier=True,
            disable_semaphore_checks=True,
            disable_bounds_checks=True,
        ),
        out_type=jax.ShapeDtypeStruct((n, DIM), jnp.float32),
        scratch_types=[
            pltpu.VMEM((rows_per_w, IDX_ROW), jnp.int32),
            pltpu.VMEM((CHUNK, DIM), jnp.float32),
            pltpu.SemaphoreType.DMA,
        ],
    )
    out = run(table, idx)
    return out.reshape(B, L, DIM)


# speedup vs baseline: 1.0486x; 1.0465x over previous
import jax
import jax.numpy as jnp
from jax import lax
from jax.experimental import pallas as pl
from jax.experimental.pallas import tpu as pltpu
from jax.experimental.pallas import tpu_sc as plsc

DIM = 64
SCALE = 8.0
LANES = 16

_info = plsc.get_sparse_core_info()
NC, NS = _info.num_cores, _info.num_subcores
NW = NC * NS

R = 4


def _emb_body(table_hbm, idx_hbm, out_hbm, idx_v, rows_v, sem):
    n_x_rows, L = idx_hbm.shape
    rows_per_w = n_x_rows // NW
    n_chunks = rows_per_w // R
    segs = []
    off = 0
    while off < L:
        w = min(128, L - off)
        segs.append((off, w))
        off += w
    wid = lax.axis_index("s") * NC + lax.axis_index("c")
    row_base = wid * rows_per_w

    pltpu.sync_copy(idx_hbm.at[pl.ds(row_base, rows_per_w)], idx_v)

    def chunk_body(ci, carry):
        row_off = ci * R
        copies = [
            pltpu.async_copy(
                table_hbm.at[idx_v.at[row_off + r, pl.ds(s_off, s_w)]],
                rows_v.at[r, pl.ds(s_off, s_w)],
                sem,
            )
            for r in range(R)
            for (s_off, s_w) in segs
        ]
        for c in copies:
            c.wait()

        def scale_body(l, acc):
            for r in range(R):
                for c0 in range(0, DIM, LANES):
                    rows_v[r, l, pl.ds(c0, LANES)] = (
                        rows_v[r, l, pl.ds(c0, LANES)] * SCALE
                    )
            return acc

        lax.fori_loop(0, L, scale_body, 0)
        pltpu.sync_copy(rows_v, out_hbm.at[pl.ds(row_base + row_off, R)])
        return carry

    lax.fori_loop(0, n_chunks, chunk_body, 0)


def kernel(x, table):
    B, L = x.shape
    rows_per_w = B // NW
    mesh = plsc.VectorSubcoreMesh(core_axis_name="c", subcore_axis_name="s")
    run = pl.kernel(
        _emb_body,
        mesh=mesh,
        compiler_params=pltpu.CompilerParams(use_tc_tiling_on_sc=False),
        out_type=jax.ShapeDtypeStruct((B, L, DIM), jnp.float32),
        scratch_types=[
            pltpu.VMEM((rows_per_w, L), jnp.int32),
            pltpu.VMEM((R, L, DIM), jnp.float32),
            pltpu.SemaphoreType.DMA,
        ],
    )
    return run(table, x)
